# baseline (speedup 1.0000x reference)
import jax
import jax.numpy as jnp
from jax.experimental import pallas as pl

N_I = 256
N_P = 8192
N_C = 9
I_BLK = 32


def _rot_plane_kernel(w_ref, x_ref, o_ref):
    w = w_ref[...]
    for d in range(3):
        acc = x_ref[0] * w[:, 3 * d : 3 * d + 1]
        acc += x_ref[1] * w[:, 3 * d + 1 : 3 * d + 2]
        acc += x_ref[2] * w[:, 3 * d + 2 : 3 * d + 3]
        o_ref[d] = acc
    for c in range(3, N_C):
        o_ref[c] = x_ref[c]


@jax.jit
def kernel(points_colored_instance, rot_mats):
    xt = jnp.transpose(points_colored_instance, (2, 0, 1))
    w = rot_mats.reshape(N_I, 9)
    out = pl.pallas_call(
        _rot_plane_kernel,
        grid=(N_I // I_BLK,),
        in_specs=[
            pl.BlockSpec((I_BLK, 9), lambda i: (i, 0)),
            pl.BlockSpec((N_C, I_BLK, N_P), lambda i: (0, i, 0)),
        ],
        out_specs=pl.BlockSpec((N_C, I_BLK, N_P), lambda i: (0, i, 0)),
        out_shape=jax.ShapeDtypeStruct((N_C, N_I, N_P), jnp.float32),
    )(w, xt)
    return jnp.transpose(out, (1, 2, 0))
